# baseline (speedup 1.0000x reference)
import functools

import jax
import jax.numpy as jnp
from jax.experimental import pallas as pl
from jax.experimental.pallas import tpu as pltpu
from jax.experimental.pallas import tpu_sc as plsc

_GATHER_WIN = 128
_VPAD = 100096
_TABLE_BLK = 4352


def _eye(k):
    r = jax.lax.broadcasted_iota(jnp.int32, (k, k), 0)
    c = jax.lax.broadcasted_iota(jnp.int32, (k, k), 1)
    return (r == c).astype(jnp.float32)


def _tc_prep_table(emb_t_pad):
    d, v = emb_t_pad.shape
    nb = v // _TABLE_BLK

    def body(in_ref, out_ref):
        blk_t = jax.lax.dot_general(
            in_ref[...], _eye(d), (((0,), (0,)), ((), ())),
            preferred_element_type=jnp.float32,
        )
        out_ref[...] = jnp.pad(blk_t, ((0, 0), (0, 128 - d)))

    return pl.pallas_call(
        body,
        grid=(nb,),
        in_specs=[pl.BlockSpec((d, _TABLE_BLK), lambda i: (0, i))],
        out_specs=pl.BlockSpec((_TABLE_BLK, 128), lambda i: (i, 0)),
        out_shape=jax.ShapeDtypeStruct((v, 128), jnp.float32),
        compiler_params=pltpu.CompilerParams(
            dimension_semantics=("parallel",),
        ),
    )(emb_t_pad)


def _sc_gather(table128, ids2d):
    n = ids2d.shape[1]
    dw = table128.shape[1]
    mesh = plsc.VectorSubcoreMesh(core_axis_name="core", subcore_axis_name="subcore")

    @functools.partial(
        pl.kernel,
        out_type=jax.ShapeDtypeStruct((n, dw), table128.dtype),
        mesh=mesh,
    )
    def gather_kernel(emb_hbm, ids_hbm, out_hbm):
        def body(i_vmem, o_vmem):
            pltpu.sync_copy(emb_hbm.at[i_vmem.at[0]], o_vmem)

        pltpu.emit_pipeline(
            body,
            grid=(n // _GATHER_WIN,),
            in_specs=[pl.BlockSpec((1, _GATHER_WIN), lambda i: (0, i))],
            out_specs=[pl.BlockSpec((_GATHER_WIN, dw), lambda i: (i, 0))],
            core_axis_name=("core", "subcore"),
            dimension_semantics=(pltpu.PARALLEL,),
        )(ids_hbm, out_hbm)

    return gather_kernel(table128, ids2d)


_SB = 4
_CHUNKS = (52, 52, 52, 44)


def _tc_combine_chunk(buf, tokens_t, pe3c, topo_t, W, c0, s_chunk):
    s, d, b = tokens_t.shape
    f = topo_t.shape[1]
    dw = pe3c.shape[2]
    cb = c0 // _SB

    def body(*refs):
        if buf is None:
            tok_ref, pe_ref, topo_ref, w_ref, out_ref = refs
        else:
            _, tok_ref, pe_ref, topo_ref, w_ref, out_ref = refs
        for j in range(_SB):
            pe_t = jax.lax.dot_general(
                _eye(d), pe_ref[j, :, :d], (((1,), (1,)), ((), ())),
                preferred_element_type=jnp.float32,
                precision=jax.lax.Precision.DEFAULT,
            )
            proj = jax.lax.dot_general(
                w_ref[...], topo_ref[j], (((1,), (0,)), ((), ())),
                preferred_element_type=jnp.float32,
            )
            out_ref[j] = tok_ref[j] + pe_t + proj

    specs = [
        pl.BlockSpec((_SB, d, b), lambda i: (cb + i, 0, 0)),
        pl.BlockSpec((_SB, b, dw), lambda i: (i, 0, 0)),
        pl.BlockSpec((_SB, f, b), lambda i: (cb + i, 0, 0)),
        pl.BlockSpec((d, f), lambda i: (0, 0)),
    ]
    args = (tokens_t, pe3c, topo_t, W)
    aliases = {}
    if buf is not None:
        specs = [pl.BlockSpec(memory_space=pl.ANY)] + specs
        args = (buf,) + args
        aliases = {0: 0}
    return pl.pallas_call(
        body,
        grid=(s_chunk // _SB,),
        in_specs=specs,
        out_specs=pl.BlockSpec((_SB, d, b), lambda i: (cb + i, 0, 0)),
        out_shape=jax.ShapeDtypeStruct((s, d, b), jnp.float32),
        input_output_aliases=aliases,
        compiler_params=pltpu.CompilerParams(
            dimension_semantics=("parallel",),
        ),
    )(*args)


def kernel(tokens, ids, topo_feats, id_emb, W_proj):
    b, s, d = tokens.shape
    n = b * s
    tokens_t = jnp.transpose(tokens, (1, 2, 0))
    topo_t = jnp.transpose(topo_feats, (1, 2, 0))
    ids_sm = ids.T.reshape(1, n).astype(jnp.int32)
    table128 = _tc_prep_table(jnp.pad(id_emb.T, ((0, 0), (0, _VPAD - id_emb.shape[0]))))
    offsets = [sum(_CHUNKS[:c]) for c in range(len(_CHUNKS))]
    pes = [
        _sc_gather(table128, ids_sm[:, o * b:(o + sc) * b]).reshape(sc, b, 128)
        for o, sc in zip(offsets, _CHUNKS)
    ]
    buf = None
    for pe, o, sc in zip(pes, offsets, _CHUNKS):
        buf = _tc_combine_chunk(buf, tokens_t, pe, topo_t, W_proj, o, sc)
    return jnp.transpose(buf, (2, 0, 1))
